# baseline (speedup 1.0000x reference)
import functools

import jax
import jax.numpy as jnp
from jax import lax
from jax.experimental import pallas as pl
from jax.experimental.pallas import tpu as pltpu
from jax.experimental.pallas import tpu_sc as plsc

MAX_CONTEXT_LENGTH = 8192
D_MODEL = 1024
BATCH = 4
SEQ_LEN = 8192

NTOT = BATCH * SEQ_LEN
NSUB = 16
CHUNK = 16
NBUF = 6
LOOKAHEAD = 3
W0 = 1120
W1 = (NTOT - NSUB * W0) // NSUB

_mesh = plsc.VectorSubcoreMesh(core_axis_name="c", subcore_axis_name="s")


@functools.partial(
    pl.kernel,
    mesh=_mesh,
    out_type=jax.ShapeDtypeStruct((NTOT, D_MODEL), jnp.float32),
    scratch_types=[
        pltpu.VMEM((max(W0, W1),), jnp.int32),
        pltpu.VMEM((NBUF, CHUNK, D_MODEL), jnp.float32),
    ]
    + [pltpu.SemaphoreType.DMA] * (2 * NBUF),
)
def _sc_gather(ids_hbm, table_hbm, out_hbm, idx_v, rows_v, *sems):
    gsem, osem = sems[:NBUF], sems[NBUF:]
    cid = lax.axis_index("c")
    sid = lax.axis_index("s")

    def pipeline(base, n_rows):
        nchunk = n_rows // CHUNK
        pltpu.sync_copy(ids_hbm.at[pl.ds(base, n_rows)], idx_v.at[pl.ds(0, n_rows)])

        def gather(g, buf):
            return pltpu.make_async_copy(
                table_hbm.at[idx_v.at[pl.ds(g * CHUNK, CHUNK)]],
                rows_v.at[buf],
                gsem[buf],
            )

        def writeback(g, buf):
            return pltpu.make_async_copy(
                rows_v.at[buf],
                out_hbm.at[pl.ds(base + g * CHUNK, CHUNK)],
                osem[buf],
            )

        for g in range(LOOKAHEAD):
            gather(g, g).start()

        def step(g, j, wait_wb, start_g):
            gather(g, j).wait()
            if start_g:
                if wait_wb:
                    writeback(
                        g + LOOKAHEAD - NBUF, (j + LOOKAHEAD) % NBUF
                    ).wait()
                gather(g + LOOKAHEAD, (j + LOOKAHEAD) % NBUF).start()
            writeback(g, j).start()

        for j in range(NBUF):
            step(j, j, j >= NBUF - LOOKAHEAD, True)

        def round_body(r, c):
            g0 = NBUF * r
            for j in range(NBUF):
                step(g0 + j, j, True, True)
            return c

        nround = nchunk // NBUF
        lax.fori_loop(1, nround, round_body, 0)

        g0 = NBUF * nround
        for j in range(nchunk - g0):
            g = g0 + j
            step(g, j, True, g + LOOKAHEAD < nchunk)
        for g in range(nchunk - NBUF, nchunk):
            writeback(g, g % NBUF).wait()

    @pl.when(cid == 0)
    def _():
        pipeline(sid * W0, W0)

    @pl.when(cid == 1)
    def _():
        pipeline(NSUB * W0 + sid * W1, W1)


def kernel(position_ids, table):
    ids_flat = position_ids.reshape(-1).astype(jnp.int32)
    out = _sc_gather(ids_flat, table)
    return out.reshape(BATCH, SEQ_LEN, D_MODEL)


# speedup vs baseline: 1.0032x; 1.0032x over previous
import functools

import jax
import jax.numpy as jnp
from jax import lax
from jax.experimental import pallas as pl
from jax.experimental.pallas import tpu as pltpu
from jax.experimental.pallas import tpu_sc as plsc

MAX_CONTEXT_LENGTH = 8192
D_MODEL = 1024
BATCH = 4
SEQ_LEN = 8192

NTOT = BATCH * SEQ_LEN
NSUB = 16
CHUNK = 16
NBUF = 6
LOOKAHEAD = 3
W0 = 1120
W1 = (NTOT - NSUB * W0) // NSUB

_mesh = plsc.VectorSubcoreMesh(core_axis_name="c", subcore_axis_name="s")


@functools.partial(
    pl.kernel,
    mesh=_mesh,
    out_type=jax.ShapeDtypeStruct((NTOT, D_MODEL), jnp.float32),
    scratch_types=[
        pltpu.VMEM((max(W0, W1),), jnp.int32),
        pltpu.VMEM((NBUF, CHUNK, D_MODEL), jnp.float32),
    ]
    + [pltpu.SemaphoreType.DMA] * (2 * NBUF),
)
def _sc_gather(ids_hbm, table_hbm, out_hbm, idx_v, rows_v, *sems):
    gsem, osem = sems[:NBUF], sems[NBUF:]
    cid = lax.axis_index("c")
    sid = lax.axis_index("s")

    def pipeline(base, n_rows):
        nchunk = n_rows // CHUNK
        pltpu.sync_copy(ids_hbm.at[pl.ds(base, n_rows)], idx_v.at[pl.ds(0, n_rows)])

        def gather(g, buf):
            return pltpu.make_async_copy(
                table_hbm.at[idx_v.at[pl.ds(g * CHUNK, CHUNK)]],
                rows_v.at[buf],
                gsem[buf],
            )

        def writeback(g, buf):
            return pltpu.make_async_copy(
                rows_v.at[buf],
                out_hbm.at[pl.ds(base + g * CHUNK, CHUNK)],
                osem[buf],
            )

        for g in range(LOOKAHEAD):
            gather(g, g).start()

        def step(g, j, wait_wb, start_g):
            gather(g, j).wait()
            if start_g:
                if wait_wb:
                    writeback(
                        g + LOOKAHEAD - NBUF, (j + LOOKAHEAD) % NBUF
                    ).wait()
                gather(g + LOOKAHEAD, (j + LOOKAHEAD) % NBUF).start()
            writeback(g, j).start()

        for j in range(NBUF):
            step(j, j, j >= NBUF - LOOKAHEAD, True)

        def round_body(r, c):
            g0 = NBUF * r
            for j in range(NBUF):
                step(g0 + j, j, True, True)
            return c

        nround = nchunk // NBUF
        lax.fori_loop(1, nround, round_body, 0)

        g0 = NBUF * nround
        for j in range(nchunk - g0):
            g = g0 + j
            step(g, j, True, g + LOOKAHEAD < nchunk)
        for g in range(nchunk - NBUF, nchunk):
            writeback(g, g % NBUF).wait()

    @pl.when(cid == 1)
    def _():
        pipeline(sid * W0, W0)

    @pl.when(cid == 0)
    def _():
        pipeline(NSUB * W0 + sid * W1, W1)


def kernel(position_ids, table):
    ids_flat = position_ids.reshape(-1).astype(jnp.int32)
    out = _sc_gather(ids_flat, table)
    return out.reshape(BATCH, SEQ_LEN, D_MODEL)


# speedup vs baseline: 1.0283x; 1.0250x over previous
import functools

import jax
import jax.numpy as jnp
from jax import lax
from jax.experimental import pallas as pl
from jax.experimental.pallas import tpu as pltpu
from jax.experimental.pallas import tpu_sc as plsc

MAX_CONTEXT_LENGTH = 8192
D_MODEL = 1024
BATCH = 4
SEQ_LEN = 8192

NTOT = BATCH * SEQ_LEN
NW = 32
B_PER_W = NTOT // NW
CHUNK = 16
NBUF = 6
LOOKAHEAD = 4
NCHUNK = B_PER_W // CHUNK

_mesh = plsc.VectorSubcoreMesh(core_axis_name="c", subcore_axis_name="s")


@functools.partial(
    pl.kernel,
    mesh=_mesh,
    out_type=jax.ShapeDtypeStruct((NTOT, D_MODEL), jnp.float32),
    scratch_types=[
        pltpu.VMEM((B_PER_W,), jnp.int32),
        pltpu.VMEM((NBUF, CHUNK, D_MODEL), jnp.float32),
    ]
    + [pltpu.SemaphoreType.DMA] * (2 * NBUF),
)
def _sc_gather(ids_hbm, table_hbm, out_hbm, idx_v, rows_v, *sems):
    gsem, osem = sems[:NBUF], sems[NBUF:]
    wid = lax.axis_index("s") * 2 + lax.axis_index("c")
    base = wid * B_PER_W
    pltpu.sync_copy(ids_hbm.at[pl.ds(base, B_PER_W)], idx_v)

    def gather(g, buf):
        return pltpu.make_async_copy(
            table_hbm.at[idx_v.at[pl.ds(g * CHUNK, CHUNK)]],
            rows_v.at[buf],
            gsem[buf],
        )

    def writeback(g, buf):
        return pltpu.make_async_copy(
            rows_v.at[buf],
            out_hbm.at[pl.ds(base + g * CHUNK, CHUNK)],
            osem[buf],
        )

    for g in range(LOOKAHEAD):
        gather(g, g).start()

    def step(g, j, wait_wb, start_g):
        gather(g, j).wait()
        if start_g:
            if wait_wb:
                writeback(g + LOOKAHEAD - NBUF, (j + LOOKAHEAD) % NBUF).wait()
            gather(g + LOOKAHEAD, (j + LOOKAHEAD) % NBUF).start()
        writeback(g, j).start()

    for j in range(NBUF):
        step(j, j, j >= NBUF - LOOKAHEAD, True)

    def round_body(r, c):
        g0 = NBUF * r
        for j in range(NBUF):
            step(g0 + j, j, True, True)
        return c

    nround = NCHUNK // NBUF
    lax.fori_loop(1, nround, round_body, 0)

    g0 = NBUF * nround
    for j in range(NCHUNK - g0):
        g = g0 + j
        step(g, j, True, g + LOOKAHEAD < NCHUNK)
    for g in range(NCHUNK - NBUF, NCHUNK):
        writeback(g, g % NBUF).wait()


def kernel(position_ids, table):
    ids_flat = position_ids.reshape(-1).astype(jnp.int32)
    out = _sc_gather(ids_flat, table)
    return out.reshape(BATCH, SEQ_LEN, D_MODEL)
